# baseline (speedup 1.0000x reference)
import functools

import jax
import jax.numpy as jnp
from jax import lax
from jax.experimental import pallas as pl
from jax.experimental.pallas import tpu as pltpu
from jax.experimental.pallas import tpu_sc as plsc

VOCAB = 100000
D = 64
B = 1024
CTX = 20

NC = 2
NS = 16
L = 16
NW = NC * NS
BPW = B // NW
IDX_PER_W = BPW * CTX
ICHUNK = 128
NCH = IDX_PER_W // ICHUNK

BM = 128
NBB = B // BM
VCH = 12544
_CHUNKS = [(s, min(VCH, VOCAB - s)) for s in range(0, VOCAB, VCH)]
VQ = 25088
NQ = 4
_QUARTERS = [
    (0, (12544, 12544)),
    (25088, (12544, 12544)),
    (50176, (12544, 12544)),
    (75264, (12544, 12192)),
]


def _sc_gather_sum(x_flat, table):
  mesh = plsc.VectorSubcoreMesh(core_axis_name="c", subcore_axis_name="s")

  @functools.partial(
      pl.kernel,
      out_type=jax.ShapeDtypeStruct((B, D), jnp.float32),
      mesh=mesh,
      scratch_types=[
          pltpu.VMEM((NCH, ICHUNK), jnp.int32),
          pltpu.VMEM((IDX_PER_W, D), jnp.float32),
          pltpu.VMEM((BPW, D), jnp.float32),
          pltpu.SemaphoreType.DMA,
          pltpu.SemaphoreType.DMA,
      ],
      compiler_params=pltpu.CompilerParams(use_tc_tiling_on_sc=False),
  )
  def k(x_hbm, tab_hbm, out_hbm, idx_v, rows_v, acc_v, isem, gsem):
    wid = lax.axis_index("s") * NC + lax.axis_index("c")
    icopies = [
        pltpu.async_copy(
            x_hbm.at[pl.ds(wid * IDX_PER_W + j * ICHUNK, ICHUNK)],
            idx_v.at[j],
            isem,
        )
        for j in range(NCH)
    ]
    for c in icopies:
      c.wait()
    copies = [
        pltpu.async_copy(
            tab_hbm.at[idx_v.at[j]],
            rows_v.at[pl.ds(j * ICHUNK, ICHUNK)],
            gsem,
        )
        for j in range(NCH)
    ]
    for c in copies:
      c.wait()

    def per_row(i, carry):
      def per_ctx(c, acc):
        r = i * CTX + c
        return tuple(acc[d] + rows_v[r, pl.ds(d * L, L)] for d in range(D // L))

      acc = lax.fori_loop(
          0, CTX, per_ctx,
          tuple(jnp.zeros((L,), jnp.float32) for _ in range(D // L)))
      for d in range(D // L):
        acc_v[i, pl.ds(d * L, L)] = acc[d]
      return carry

    lax.fori_loop(0, BPW, per_row, 0)
    pltpu.sync_copy(acc_v, out_hbm.at[pl.ds(wid * BPW, BPW)])

  return k(x_flat, table)


def _fused_body(s_ref, wt_ref, b_ref, o_ref, lse_ref):
  p = pl.program_id(1)
  s = s_ref[...]

  def _logits(st, w):
    lg = lax.dot_general(
        s, wt_ref[:, slice(st, st + w)], (((1,), (0,)), ((), ())),
        preferred_element_type=jnp.float32)
    return lg + b_ref[:, slice(st, st + w)]

  @pl.when(p == 0)
  def _():
    m = jnp.full((BM, 1), -jnp.inf, jnp.float32)
    l = jnp.zeros((BM, 1), jnp.float32)
    for (st, w) in _CHUNKS:
      lg = _logits(st, w)
      m_new = jnp.maximum(m, jnp.max(lg, axis=1, keepdims=True))
      e = jnp.exp(lg - m_new)
      psum = lax.dot_general(
          e, jnp.ones((w, 1), jnp.float32), (((1,), (0,)), ((), ())),
          preferred_element_type=jnp.float32)
      l = l * jnp.exp(m - m_new) + psum
      m = m_new
    lse_ref[...] = m + jnp.log(l)

  for k, (qoff, widths) in enumerate(_QUARTERS):
    @pl.when(p == k + 1)
    def _(qoff=qoff, widths=widths):
      lse = lse_ref[...]
      off = 0
      for w in widths:
        o_ref[:, slice(off, off + w)] = _logits(qoff + off, w) - lse
        off += w


def kernel(x, embedding_matrix, W, b):
  x_flat = x.astype(jnp.int32).reshape(B * CTX)
  summed = _sc_gather_sum(x_flat, embedding_matrix)
  wt = W.T
  b2 = b.reshape(1, VOCAB)

  out = pl.pallas_call(
      _fused_body,
      grid=(NBB, NQ + 1),
      in_specs=[
          pl.BlockSpec((BM, D), lambda i, p: (i, 0)),
          pl.BlockSpec((D, VOCAB), lambda i, p: (0, 0)),
          pl.BlockSpec((1, VOCAB), lambda i, p: (0, 0)),
      ],
      out_specs=pl.BlockSpec((BM, VQ), lambda i, p: (i, jnp.maximum(p - 1, 0))),
      out_shape=jax.ShapeDtypeStruct((B, VOCAB), jnp.float32),
      scratch_shapes=[pltpu.VMEM((BM, 1), jnp.float32)],
      compiler_params=pltpu.CompilerParams(
          dimension_semantics=("arbitrary", "arbitrary")),
  )(summed, wt, b2)
  return out
